# Initial kernel scaffold; baseline (speedup 1.0000x reference)
import functools

import jax
import jax.numpy as jnp
from jax import lax
from jax.experimental import pallas as pl
from jax.experimental.pallas import tpu as pltpu
from jax.experimental.pallas import tpu_sc as plsc

NC = 2
NS = 16
NW = NC * NS
EB = 80
RB = 80


def _sc_accumulate(N, d, E, IP):
  assert E % (NW * EB) == 0
  nb = E // (NW * EB)
  assert IP % (NS * RB) == 0
  rb_per_tile = IP // (NS * RB)
  zrows = 125
  rows_per_tile = N // NS
  assert rows_per_tile % zrows == 0
  cnt_chunk = N // NS
  assert cnt_chunk % d == 0

  mesh = plsc.VectorSubcoreMesh(core_axis_name="c", subcore_axis_name="s",
                                num_cores=NC, num_subcores=NS)

  @functools.partial(
      pl.kernel,
      mesh=mesh,
      out_type=[
          jax.ShapeDtypeStruct((NC, IP, d), jnp.float32),
          jax.ShapeDtypeStruct((NC, IP), jnp.float32),
          jax.ShapeDtypeStruct((IP, d), jnp.float32),
      ],
      scratch_types=[
          pltpu.VMEM((EB,), jnp.int32),
          pltpu.VMEM((EB,), jnp.int32),
          pltpu.VMEM((RB,), jnp.int32),
          pltpu.VMEM((EB,), jnp.float32),
          pltpu.VMEM((EB, d), jnp.float32),
          pltpu.VMEM((RB, d), jnp.float32),
          pltpu.VMEM((RB,), jnp.float32),
          pltpu.VMEM((125, d), jnp.float32),
          pltpu.VMEM_SHARED((N, d), jnp.float32),
          pltpu.VMEM_SHARED((N,), jnp.float32),
      ],
  )
  def k(x_hbm, src_hbm, dst_hbm, idx_hbm, g_hbm, c_hbm, xg_hbm,
        srcbuf, dstbuf, ibuf, ones, erows, grows, cbuf, zbuf, acc_sh, cnt_sh):
    cid = lax.axis_index("c")
    sid = lax.axis_index("s")
    wid = sid * NC + cid

    def zero_row(r, _):
      for kk in range(d // 16):
        zbuf[r, pl.ds(kk * 16, 16)] = jnp.zeros((16,), jnp.float32)
      return 0
    lax.fori_loop(0, zrows, zero_row, 0)
    for kk in range(EB // 16):
      ones[pl.ds(kk * 16, 16)] = jnp.ones((16,), jnp.float32)

    for z in range(rows_per_tile // zrows):
      pltpu.sync_copy(zbuf, acc_sh.at[pl.ds(sid * rows_per_tile + z * zrows,
                                            zrows), :])
    for z in range(cnt_chunk // d):
      pltpu.sync_copy(zbuf.at[0], cnt_sh.at[pl.ds(sid * cnt_chunk + z * d, d)])
    plsc.subcore_barrier()

    e_base = wid * (nb * EB)

    def edge_batch(it, _):
      be = pl.multiple_of(e_base + it * EB, 8)
      pltpu.sync_copy(src_hbm.at[pl.ds(be, EB)], srcbuf)
      pltpu.sync_copy(dst_hbm.at[pl.ds(be, EB)], dstbuf)
      pltpu.sync_copy(x_hbm.at[srcbuf], erows)
      pltpu.sync_copy(erows, acc_sh.at[dstbuf], add=True)
      pltpu.sync_copy(ones, cnt_sh.at[dstbuf], add=True)
      return 0
    lax.fori_loop(0, nb, edge_batch, 0)
    plsc.subcore_barrier()

    for j in range(rb_per_tile):
      rbase = pl.multiple_of(sid * (rb_per_tile * RB) + j * RB, 8)
      pltpu.sync_copy(idx_hbm.at[pl.ds(rbase, RB)], ibuf)
      pltpu.sync_copy(acc_sh.at[ibuf], grows)
      pltpu.sync_copy(grows, g_hbm.at[cid, pl.ds(rbase, RB)])
      pltpu.sync_copy(cnt_sh.at[ibuf], cbuf)
      pltpu.sync_copy(cbuf, c_hbm.at[cid, pl.ds(rbase, RB)])

    @pl.when(cid == 0)
    def _():
      for j in range(rb_per_tile):
        rbase = pl.multiple_of(sid * (rb_per_tile * RB) + j * RB, 8)
        pltpu.sync_copy(idx_hbm.at[pl.ds(rbase, RB)], ibuf)
        pltpu.sync_copy(x_hbm.at[ibuf], grows)
        pltpu.sync_copy(grows, xg_hbm.at[pl.ds(rbase, RB)])

  return k


def _tc_finish(IP, d, RBLK):
  assert IP % RBLK == 0

  def body(g0, g1, xg, c0, c1, w, b, out):
    s = g0[...] + g1[...] + xg[...]
    c = c0[...] + c1[...] + 1.0
    m = s / c
    out[...] = lax.dot_general(
        m, w[...], dimension_numbers=(((1,), (1,)), ((), ())),
        preferred_element_type=jnp.float32) + b[...]

  return pl.pallas_call(
      body,
      grid=(IP // RBLK,),
      in_specs=[
          pl.BlockSpec((RBLK, d), lambda i: (i, 0)),
          pl.BlockSpec((RBLK, d), lambda i: (i, 0)),
          pl.BlockSpec((RBLK, d), lambda i: (i, 0)),
          pl.BlockSpec((RBLK, 1), lambda i: (i, 0)),
          pl.BlockSpec((RBLK, 1), lambda i: (i, 0)),
          pl.BlockSpec((d, d), lambda i: (0, 0)),
          pl.BlockSpec((1, d), lambda i: (0, 0)),
      ],
      out_specs=pl.BlockSpec((RBLK, d), lambda i: (i, 0)),
      out_shape=jax.ShapeDtypeStruct((IP, d), jnp.float32),
  )


def kernel(x, edge_index, idx, W, b):
  N, d = x.shape
  E = edge_index.shape[1]
  Nsel = idx.shape[0]
  IP = ((Nsel + NS * RB - 1) // (NS * RB)) * (NS * RB)

  src = edge_index[0].astype(jnp.int32)
  dst = edge_index[1].astype(jnp.int32)
  idx32 = idx.astype(jnp.int32)
  idxp = jnp.pad(idx32, (0, IP - Nsel))

  g, c, xg = _sc_accumulate(N, d, E, IP)(x, src, dst, idxp)
  out_full = _tc_finish(IP, d, 640)(
      g[0], g[1], xg, c[0].reshape(IP, 1), c[1].reshape(IP, 1),
      W, b.reshape(1, d))
  return (out_full[:Nsel], idx)


# speedup vs baseline: 8.0851x; 8.0851x over previous
import functools

import jax
import jax.numpy as jnp
from jax import lax
from jax.experimental import pallas as pl
from jax.experimental.pallas import tpu as pltpu
from jax.experimental.pallas import tpu_sc as plsc

NC = 2
NS = 16
NW = NC * NS
EB = 80
RB = 80


def _sc_accumulate(N, d, E, IP):
  assert E % (NW * EB) == 0
  nb = E // (NW * EB)
  assert IP % (NS * RB) == 0
  rb_per_tile = IP // (NS * RB)
  zrows = 125
  rows_per_tile = N // NS
  assert rows_per_tile % zrows == 0
  zcnt_len = 2000
  assert N % zcnt_len == 0 and N // zcnt_len <= NS

  mesh = plsc.VectorSubcoreMesh(core_axis_name="c", subcore_axis_name="s",
                                num_cores=NC, num_subcores=NS)

  @functools.partial(
      pl.kernel,
      mesh=mesh,
      out_type=[
          jax.ShapeDtypeStruct((NC, IP, d), jnp.float32),
          jax.ShapeDtypeStruct((NC * IP,), jnp.float32),
          jax.ShapeDtypeStruct((IP, d), jnp.float32),
      ],
      scratch_types=[
          pltpu.VMEM((EB,), jnp.int32),
          pltpu.VMEM((EB,), jnp.int32),
          pltpu.VMEM((RB,), jnp.int32),
          pltpu.VMEM((EB,), jnp.float32),
          pltpu.VMEM((EB, d), jnp.float32),
          pltpu.VMEM((RB, d), jnp.float32),
          pltpu.VMEM((RB,), jnp.float32),
          pltpu.VMEM((125, d), jnp.float32),
          pltpu.VMEM((zcnt_len,), jnp.float32),
          pltpu.VMEM_SHARED((N, d), jnp.float32),
          pltpu.VMEM_SHARED((N,), jnp.float32),
      ],
  )
  def k(x_hbm, src_hbm, dst_hbm, idx_hbm, g_hbm, c_hbm, xg_hbm,
        srcbuf, dstbuf, ibuf, ones, erows, grows, cbuf, zbuf, zcnt,
        acc_sh, cnt_sh):
    cid = lax.axis_index("c")
    sid = lax.axis_index("s")
    wid = sid * NC + cid

    def zero_row(r, _):
      for kk in range(d // 16):
        zbuf[r, pl.ds(kk * 16, 16)] = jnp.zeros((16,), jnp.float32)
      return 0
    lax.fori_loop(0, zrows, zero_row, 0)
    for kk in range(EB // 16):
      ones[pl.ds(kk * 16, 16)] = jnp.ones((16,), jnp.float32)
    def zero_cnt(r, _):
      zcnt[pl.ds(r * 16, 16)] = jnp.zeros((16,), jnp.float32)
      return 0
    lax.fori_loop(0, zcnt_len // 16, zero_cnt, 0)

    for z in range(rows_per_tile // zrows):
      pltpu.sync_copy(zbuf, acc_sh.at[pl.ds(sid * rows_per_tile + z * zrows,
                                            zrows), :])
    @pl.when(sid < N // zcnt_len)
    def _():
      pltpu.sync_copy(zcnt, cnt_sh.at[pl.ds(sid * zcnt_len, zcnt_len)])
    plsc.subcore_barrier()

    e_base = wid * (nb * EB)

    def edge_batch(it, _):
      be = pl.multiple_of(e_base + it * EB, 8)
      pltpu.sync_copy(src_hbm.at[pl.ds(be, EB)], srcbuf)
      pltpu.sync_copy(dst_hbm.at[pl.ds(be, EB)], dstbuf)
      pltpu.sync_copy(x_hbm.at[srcbuf], erows)
      pltpu.sync_copy(erows, acc_sh.at[dstbuf], add=True)
      pltpu.sync_copy(ones, cnt_sh.at[dstbuf], add=True)
      return 0
    lax.fori_loop(0, nb, edge_batch, 0)
    plsc.subcore_barrier()

    for j in range(rb_per_tile):
      rbase = pl.multiple_of(sid * (rb_per_tile * RB) + j * RB, 8)
      pltpu.sync_copy(idx_hbm.at[pl.ds(rbase, RB)], ibuf)
      pltpu.sync_copy(acc_sh.at[ibuf], grows)
      pltpu.sync_copy(grows, g_hbm.at[cid, pl.ds(rbase, RB)])
      pltpu.sync_copy(cnt_sh.at[ibuf], cbuf)
      cofs = pl.multiple_of(cid * IP + rbase, 8)
      pltpu.sync_copy(cbuf, c_hbm.at[pl.ds(cofs, RB)])

    @pl.when(cid == 0)
    def _():
      for j in range(rb_per_tile):
        rbase = pl.multiple_of(sid * (rb_per_tile * RB) + j * RB, 8)
        pltpu.sync_copy(idx_hbm.at[pl.ds(rbase, RB)], ibuf)
        pltpu.sync_copy(x_hbm.at[ibuf], grows)
        pltpu.sync_copy(grows, xg_hbm.at[pl.ds(rbase, RB)])

  return k


def _tc_finish(IP, d, RBLK):
  assert IP % RBLK == 0

  def body(g0, g1, xg, c0, c1, w, b, out):
    s = g0[...] + g1[...] + xg[...]
    c = c0[...] + c1[...] + 1.0
    m = s / c
    out[...] = lax.dot_general(
        m, w[...], dimension_numbers=(((1,), (1,)), ((), ())),
        preferred_element_type=jnp.float32) + b[...]

  return pl.pallas_call(
      body,
      grid=(IP // RBLK,),
      in_specs=[
          pl.BlockSpec((RBLK, d), lambda i: (i, 0)),
          pl.BlockSpec((RBLK, d), lambda i: (i, 0)),
          pl.BlockSpec((RBLK, d), lambda i: (i, 0)),
          pl.BlockSpec((RBLK, 1), lambda i: (i, 0)),
          pl.BlockSpec((RBLK, 1), lambda i: (i, 0)),
          pl.BlockSpec((d, d), lambda i: (0, 0)),
          pl.BlockSpec((1, d), lambda i: (0, 0)),
      ],
      out_specs=pl.BlockSpec((RBLK, d), lambda i: (i, 0)),
      out_shape=jax.ShapeDtypeStruct((IP, d), jnp.float32),
  )


def kernel(x, edge_index, idx, W, b):
  N, d = x.shape
  E = edge_index.shape[1]
  Nsel = idx.shape[0]
  IP = ((Nsel + NS * RB - 1) // (NS * RB)) * (NS * RB)

  src = edge_index[0].astype(jnp.int32)
  dst = edge_index[1].astype(jnp.int32)
  idx32 = idx.astype(jnp.int32)
  idxp = jnp.pad(idx32, (0, IP - Nsel))

  g, c, xg = _sc_accumulate(N, d, E, IP)(x, src, dst, idxp)
  out_full = _tc_finish(IP, d, 640)(
      g[0], g[1], xg, c[:IP].reshape(IP, 1), c[IP:].reshape(IP, 1),
      W, b.reshape(1, d))
  return (out_full[:Nsel], idx)


# speedup vs baseline: 9.9468x; 1.2303x over previous
import functools

import jax
import jax.numpy as jnp
from jax import lax
from jax.experimental import pallas as pl
from jax.experimental.pallas import tpu as pltpu
from jax.experimental.pallas import tpu_sc as plsc

NC = 2
NS = 16
NW = NC * NS
EB = 112
RB = 112
DEPTH = 2
ZR = 1000


def _sc_accumulate(N, d, E2, IP, NPAD):
  assert E2 % (NW * EB) == 0
  nb = E2 // (NW * EB)
  assert nb % DEPTH == 0
  assert IP % (NS * RB) == 0
  rb_per_tile = IP // (NS * RB)
  assert N % ZR == 0 and N // ZR <= NS

  mesh = plsc.VectorSubcoreMesh(core_axis_name="c", subcore_axis_name="s",
                                num_cores=NC, num_subcores=NS)

  @functools.partial(
      pl.kernel,
      mesh=mesh,
      out_type=[
          jax.ShapeDtypeStruct((NC, IP, d), jnp.float32),
          jax.ShapeDtypeStruct((NC * IP,), jnp.float32),
          jax.ShapeDtypeStruct((IP, d), jnp.float32),
      ],
      scratch_types=[
          pltpu.VMEM((nb, EB), jnp.int32),
          [pltpu.VMEM((EB,), jnp.int32) for _ in range(DEPTH)],
          pltpu.VMEM((RB,), jnp.int32),
          pltpu.VMEM((EB,), jnp.float32),
          [pltpu.VMEM((EB, d), jnp.float32) for _ in range(DEPTH)],
          pltpu.VMEM((RB,), jnp.float32),
          pltpu.VMEM((ZR,), jnp.float32),
          pltpu.VMEM_SHARED((NPAD, d), jnp.float32),
          pltpu.VMEM_SHARED((NPAD,), jnp.float32),
          [pltpu.SemaphoreType.DMA for _ in range(DEPTH)],
          [pltpu.SemaphoreType.DMA for _ in range(DEPTH)],
          [pltpu.SemaphoreType.DMA for _ in range(DEPTH)],
          [pltpu.SemaphoreType.DMA for _ in range(DEPTH)],
      ],
  )
  def k(x_hbm, srcm_hbm, dstm_hbm, idx_hbm, zr_hbm, zc_hbm,
        g_hbm, c_hbm, xg_hbm,
        srcc, dstbuf, ibuf, ones, erows, cbuf, zcv,
        acc_sh, cnt_sh, gsem, ssem, csem, isem):
    cid = lax.axis_index("c")
    sid = lax.axis_index("s")
    wid = sid * NC + cid

    pltpu.sync_copy(srcm_hbm.at[wid], srcc)
    for kk in range(EB // 16):
      ones[pl.ds(kk * 16, 16)] = jnp.ones((16,), jnp.float32)

    @pl.when(sid < N // ZR)
    def _():
      pltpu.sync_copy(zr_hbm, acc_sh.at[pl.ds(sid * ZR, ZR), :])
      pltpu.sync_copy(zc_hbm, zcv)
      pltpu.sync_copy(zcv, cnt_sh.at[pl.ds(sid * ZR, ZR)])
    plsc.subcore_barrier()

    for b in range(DEPTH):
      pltpu.async_copy(dstm_hbm.at[wid, b], dstbuf[b], isem[b])
      pltpu.async_copy(x_hbm.at[srcc.at[b]], erows[b], gsem[b])

    def row_round(o, _):
      for b in range(DEPTH):
        it = o * DEPTH + b
        pltpu.make_async_copy(x_hbm.at[srcc.at[it]], erows[b], gsem[b]).wait()
        pltpu.make_async_copy(dstm_hbm.at[wid, it], dstbuf[b], isem[b]).wait()
        pltpu.async_copy(erows[b], acc_sh.at[dstbuf[b]], ssem[b], add=True)
        pltpu.async_copy(ones, cnt_sh.at[dstbuf[b]], csem[b], add=True)

        @pl.when(o < nb // DEPTH - 1)
        def _():
          pltpu.make_async_copy(erows[b], acc_sh.at[dstbuf[b]],
                                ssem[b]).wait()
          pltpu.make_async_copy(ones, cnt_sh.at[dstbuf[b]], csem[b]).wait()
          pltpu.async_copy(dstm_hbm.at[wid, it + DEPTH], dstbuf[b], isem[b])
          pltpu.async_copy(x_hbm.at[srcc.at[it + DEPTH]], erows[b], gsem[b])
      return 0
    lax.fori_loop(0, nb // DEPTH, row_round, 0)

    for b in range(DEPTH):
      pltpu.make_async_copy(erows[b], acc_sh.at[dstbuf[b]], ssem[b]).wait()
      pltpu.make_async_copy(ones, cnt_sh.at[dstbuf[b]], csem[b]).wait()
    plsc.subcore_barrier()

    for j in range(rb_per_tile):
      rbase = pl.multiple_of(sid * (rb_per_tile * RB) + j * RB, 8)
      pltpu.sync_copy(idx_hbm.at[pl.ds(rbase, RB)], ibuf)
      pltpu.sync_copy(acc_sh.at[ibuf], erows[0])
      pltpu.sync_copy(erows[0], g_hbm.at[cid, pl.ds(rbase, RB)])
      pltpu.sync_copy(cnt_sh.at[ibuf], cbuf)
      cofs = pl.multiple_of(cid * IP + rbase, 8)
      pltpu.sync_copy(cbuf, c_hbm.at[pl.ds(cofs, RB)])

    @pl.when(cid == 0)
    def _():
      for j in range(rb_per_tile):
        rbase = pl.multiple_of(sid * (rb_per_tile * RB) + j * RB, 8)
        pltpu.sync_copy(idx_hbm.at[pl.ds(rbase, RB)], ibuf)
        pltpu.sync_copy(x_hbm.at[ibuf], erows[1])
        pltpu.sync_copy(erows[1], xg_hbm.at[pl.ds(rbase, RB)])

  return k


def _tc_finish(IP, d, RBLK):
  assert IP % RBLK == 0

  def body(g0, g1, xg, c0, c1, w, b, out):
    s = g0[...] + g1[...] + xg[...]
    c = c0[...] + c1[...] + 1.0
    m = s / c
    out[...] = lax.dot_general(
        m, w[...], dimension_numbers=(((1,), (1,)), ((), ())),
        preferred_element_type=jnp.float32) + b[...]

  return pl.pallas_call(
      body,
      grid=(IP // RBLK,),
      in_specs=[
          pl.BlockSpec((RBLK, d), lambda i: (i, 0)),
          pl.BlockSpec((RBLK, d), lambda i: (i, 0)),
          pl.BlockSpec((RBLK, d), lambda i: (i, 0)),
          pl.BlockSpec((RBLK, 1), lambda i: (i, 0)),
          pl.BlockSpec((RBLK, 1), lambda i: (i, 0)),
          pl.BlockSpec((d, d), lambda i: (0, 0)),
          pl.BlockSpec((1, d), lambda i: (0, 0)),
      ],
      out_specs=pl.BlockSpec((RBLK, d), lambda i: (i, 0)),
      out_shape=jax.ShapeDtypeStruct((IP, d), jnp.float32),
  )


def kernel(x, edge_index, idx, W, b):
  N, d = x.shape
  E = edge_index.shape[1]
  Nsel = idx.shape[0]
  IP = ((Nsel + NS * RB - 1) // (NS * RB)) * (NS * RB)
  EQ = NW * EB * DEPTH
  E2 = ((E + EQ - 1) // EQ) * EQ
  NPAD = N + 8

  src = edge_index[0].astype(jnp.int32)
  dst = edge_index[1].astype(jnp.int32)
  nb = E2 // (NW * EB)
  srcm = jnp.pad(src, (0, E2 - E)).reshape(NW, nb, EB)
  dstm = jnp.pad(dst, (0, E2 - E), constant_values=N).reshape(NW, nb, EB)
  idx32 = idx.astype(jnp.int32)
  idxp = jnp.pad(idx32, (0, IP - Nsel))
  zrows = jnp.zeros((ZR, d), jnp.float32)
  zcnt = jnp.zeros((ZR,), jnp.float32)

  g, c, xg = _sc_accumulate(N, d, E2, IP, NPAD)(
      x, srcm, dstm, idxp, zrows, zcnt)
  out_full = _tc_finish(IP, d, 448)(
      g[0], g[1], xg, c[:IP].reshape(IP, 1), c[IP:].reshape(IP, 1),
      W, b.reshape(1, d))
  return (out_full[:Nsel], idx)
